# baseline (speedup 1.0000x reference)
import functools

import jax
import jax.numpy as jnp
from jax import lax
from jax.experimental import pallas as pl
from jax.experimental.pallas import tpu as pltpu
from jax.experimental.pallas import tpu_sc as plsc

N = 10000
E = 320000
D = 128
DOUT = 64
NG = 64
DEPTH = 3
EPS = 1e-5

NC, NS = 2, 16
NW = NC * NS
EPW = E // NW
CK = 80
NCH = EPW // CK
NP = 10112
RPT = NP // NS

RB = 1000
NRB = N // RB


@functools.cache
def _mesh():
    return plsc.VectorSubcoreMesh(
        core_axis_name="c", subcore_axis_name="s",
        num_cores=NC, num_subcores=NS)


def _edge_body(g_hbm, src_hbm, dst_hbm, accp_hbm, srcv, dstv, rows, acc,
               sem):
    c = lax.axis_index("c")
    s = lax.axis_index("s")
    w = c * NS + s

    @pl.loop(0, CK)
    def _z(i):
        for j in range(D // 16):
            rows[i, pl.ds(j * 16, 16)] = jnp.zeros((16,), jnp.float32)

    for t in range(RPT // CK):
        pltpu.sync_copy(rows, acc.at[pl.ds(s * RPT + t * CK, CK)])
    pltpu.sync_copy(rows.at[pl.ds(0, RPT % CK)],
                    acc.at[pl.ds(s * RPT + RPT - RPT % CK, RPT % CK)])
    plsc.subcore_barrier()
    pltpu.sync_copy(src_hbm.at[w], srcv)
    pltpu.sync_copy(dst_hbm.at[w], dstv)

    @pl.loop(0, NCH)
    def _go(i):
        pltpu.async_copy(g_hbm.at[srcv.at[i]], rows, sem).wait()
        pltpu.sync_copy(rows, acc.at[dstv.at[i]], add=True)

    plsc.subcore_barrier()
    for t in range(RPT // CK):
        pltpu.sync_copy(acc.at[pl.ds(s * RPT + t * CK, CK)], rows)
        pltpu.sync_copy(rows, accp_hbm.at[c, pl.ds(s * RPT + t * CK, CK)])
    pltpu.sync_copy(acc.at[pl.ds(s * RPT + RPT - RPT % CK, RPT % CK)],
                    rows.at[pl.ds(0, RPT % CK)])
    pltpu.sync_copy(rows.at[pl.ds(0, RPT % CK)],
                    accp_hbm.at[c, pl.ds(s * RPT + RPT - RPT % CK, RPT % CK)])


@functools.cache
def _edge_call():
    return pl.kernel(
        _edge_body,
        out_type=jax.ShapeDtypeStruct((NC, NP, D), jnp.float32),
        mesh=_mesh(),
        scratch_types=[
            pltpu.VMEM((NCH, CK), jnp.int32),
            pltpu.VMEM((NCH, CK), jnp.int32),
            pltpu.VMEM((CK, D), jnp.float32),
            pltpu.VMEM_SHARED((NP, D), jnp.float32),
            pltpu.SemaphoreType.DMA,
        ],
    )


def _k1_body(x_ref, degp_ref, w1, b1, w2, b2, wc, g_ref, dinv_ref):
    h = jnp.maximum(
        jnp.dot(x_ref[...], w1[...], preferred_element_type=jnp.float32)
        + b1[...], 0.0)
    h = jnp.dot(h, w2[...], preferred_element_type=jnp.float32) + b2[...]
    deg = degp_ref[0, :, 0:1] + degp_ref[1, :, 0:1] + 1.0
    dinv = lax.rsqrt(deg)
    dinv_ref[...] = dinv
    g_ref[...] = jnp.dot(h, wc[...], preferred_element_type=jnp.float32) * dinv


def _full(shape):
    return pl.BlockSpec(shape, lambda *_: tuple(0 for _ in shape))


_k1_call = pl.pallas_call(
    _k1_body,
    grid=(NRB,),
    in_specs=[
        pl.BlockSpec((RB, D), lambda i: (i, 0)),
        pl.BlockSpec((NC, RB, D), lambda i: (0, i, 0)),
        _full((D, D)),
        _full((1, D)),
        _full((D, D)),
        _full((1, D)),
        _full((D, D)),
    ],
    out_specs=[
        pl.BlockSpec((RB, D), lambda i: (i, 0)),
        pl.BlockSpec((RB, 1), lambda i: (i, 0)),
    ],
    out_shape=[
        jax.ShapeDtypeStruct((N, D), jnp.float32),
        jax.ShapeDtypeStruct((N, 1), jnp.float32),
    ],
)


def _k2_body(accp_ref, g_ref, dinv_ref, bc, gamma, beta, wn, out_ref, zs,
             stats):
    p = pl.program_id(0)
    i = pl.program_id(1)
    dinv = dinv_ref[...]

    @pl.when(p == 0)
    def _pass0():
        z = dinv * (accp_ref[0] + accp_ref[1] + g_ref[...]) + bc[...]
        zs[pl.ds(i * RB, RB)] = z

        @pl.when(i == 0)
        def _init():
            stats[...] = jnp.zeros_like(stats)

        stats[0:1] = stats[0:1] + jnp.sum(z, axis=0, keepdims=True)
        stats[1:2] = stats[1:2] + jnp.sum(z * z, axis=0, keepdims=True)

    @pl.when(p == 1)
    def _pass1():
        z = zs[pl.ds(i * RB, RB)]
        mean = stats[0:1] / N
        var = stats[1:2] / N - mean * mean
        hn = jnp.maximum(
            (z - mean) * lax.rsqrt(var + EPS) * gamma[...] + beta[...], 0.0)
        out_ref[...] = jnp.dot(
            hn, wn[...], preferred_element_type=jnp.float32) * dinv


_k2_call = pl.pallas_call(
    _k2_body,
    grid=(2, NRB),
    in_specs=[
        pl.BlockSpec((NC, RB, D), lambda p, i: (0, i, 0)),
        pl.BlockSpec((RB, D), lambda p, i: (i, 0)),
        pl.BlockSpec((RB, 1), lambda p, i: (i, 0)),
        _full((1, D)),
        _full((1, D)),
        _full((1, D)),
        _full((D, D)),
    ],
    out_specs=pl.BlockSpec((RB, D), lambda p, i: (i, 0)),
    out_shape=jax.ShapeDtypeStruct((N, D), jnp.float32),
    scratch_shapes=[
        pltpu.VMEM((N, D), jnp.float32),
        pltpu.VMEM((2, D), jnp.float32),
    ],
)


def _k3_body(accp_ref, g_ref, dinv_ref, batch_ref, bc, gamma, beta, p1, pb1,
             p2, pb2, y_ref, zs, stats):
    p = pl.program_id(0)
    i = pl.program_id(1)
    dinv = dinv_ref[...]

    @pl.when(p == 0)
    def _pass0():
        z = dinv * (accp_ref[0] + accp_ref[1] + g_ref[...]) + bc[...]
        zs[pl.ds(i * RB, RB)] = z

        @pl.when(i == 0)
        def _init():
            stats[...] = jnp.zeros_like(stats)
            y_ref[...] = jnp.zeros_like(y_ref)

        stats[0:1] = stats[0:1] + jnp.sum(z, axis=0, keepdims=True)
        stats[1:2] = stats[1:2] + jnp.sum(z * z, axis=0, keepdims=True)

    @pl.when(p == 1)
    def _pass1():
        z = zs[pl.ds(i * RB, RB)]
        mean = stats[0:1] / N
        var = stats[1:2] / N - mean * mean
        hn = jnp.maximum(
            (z - mean) * lax.rsqrt(var + EPS) * gamma[...] + beta[...], 0.0)
        t = jnp.maximum(
            jnp.dot(hn, p1[...], preferred_element_type=jnp.float32)
            + pb1[...], 0.0)
        o = jnp.dot(t, p2[...], preferred_element_type=jnp.float32) + pb2[...]
        gids = lax.broadcasted_iota(jnp.int32, (1, NG), 1)
        mask = (batch_ref[...] == gids).astype(jnp.float32)
        y_ref[...] += lax.dot_general(
            mask, o, (((0,), (0,)), ((), ())),
            preferred_element_type=jnp.float32)


_k3_call = pl.pallas_call(
    _k3_body,
    grid=(2, NRB),
    in_specs=[
        pl.BlockSpec((NC, RB, D), lambda p, i: (0, i, 0)),
        pl.BlockSpec((RB, D), lambda p, i: (i, 0)),
        pl.BlockSpec((RB, 1), lambda p, i: (i, 0)),
        pl.BlockSpec((RB, 1), lambda p, i: (i, 0)),
        _full((1, D)),
        _full((1, D)),
        _full((1, D)),
        _full((D, DOUT)),
        _full((1, DOUT)),
        _full((DOUT, DOUT)),
        _full((1, DOUT)),
    ],
    out_specs=_full((NG, DOUT)),
    out_shape=jax.ShapeDtypeStruct((NG, DOUT), jnp.float32),
    scratch_shapes=[
        pltpu.VMEM((N, D), jnp.float32),
        pltpu.VMEM((2, D), jnp.float32),
    ],
)


def kernel(x, edge_index, batch, weights):
    w = weights
    W1, b1, W2, b2 = w[0], w[1], w[2], w[3]
    convs = [(w[4 + 4 * l], w[5 + 4 * l], w[6 + 4 * l], w[7 + 4 * l])
             for l in range(DEPTH)]
    P1, pb1, P2, pb2 = w[16], w[17], w[18], w[19]

    src = edge_index[0].reshape(NW, NCH, CK)
    dst = edge_index[1].reshape(NW, NCH, CK)
    zsrc = jnp.zeros((NW, NCH, CK), jnp.int32)
    ones = jnp.ones((N, D), jnp.float32)

    r = lambda v: v.reshape(1, -1)

    degp = _edge_call()(ones, zsrc, dst)
    g, dinv = _k1_call(x, degp, W1, r(b1), W2, r(b2), convs[0][0])
    for l in range(DEPTH):
        Wc, bc, gamma, beta = convs[l]
        accp = _edge_call()(g, src, dst)
        if l < DEPTH - 1:
            g = _k2_call(accp, g, dinv, r(bc), r(gamma), r(beta),
                         convs[l + 1][0])
        else:
            y = _k3_call(accp, g, dinv, batch.reshape(N, 1), r(bc), r(gamma),
                         r(beta), P1, r(pb1), P2, r(pb2))
    return y


# speedup vs baseline: 13.6901x; 13.6901x over previous
import functools

import jax
import jax.numpy as jnp
from jax import lax
from jax.experimental import pallas as pl
from jax.experimental.pallas import tpu as pltpu
from jax.experimental.pallas import tpu_sc as plsc

N = 10000
E = 320000
D = 128
DOUT = 64
NG = 64
DEPTH = 3
EPS = 1e-5

NC, NS = 2, 16
NW = NC * NS
EPW = E // NW
CK = 80
NCH = EPW // CK
NP = 10112
RPT = NP // NS

RB = 1000
NRB = N // RB


@functools.cache
def _mesh():
    return plsc.VectorSubcoreMesh(
        core_axis_name="c", subcore_axis_name="s",
        num_cores=NC, num_subcores=NS)


def _edge_body(g_hbm, src_hbm, dst_hbm, accp_hbm, srcv, dstv, rows, acc,
               sem):
    c = lax.axis_index("c")
    s = lax.axis_index("s")
    w = c * NS + s

    @pl.loop(0, CK)
    def _z(i):
        for j in range(D // 16):
            rows[i, pl.ds(j * 16, 16)] = jnp.zeros((16,), jnp.float32)

    for t in range(RPT // CK):
        pltpu.sync_copy(rows, acc.at[pl.ds(s * RPT + t * CK, CK)])
    pltpu.sync_copy(rows.at[pl.ds(0, RPT % CK)],
                    acc.at[pl.ds(s * RPT + RPT - RPT % CK, RPT % CK)])
    plsc.subcore_barrier()
    pltpu.sync_copy(src_hbm.at[w], srcv)
    pltpu.sync_copy(dst_hbm.at[w], dstv)

    @pl.loop(0, NCH)
    def _go(i):
        pltpu.async_copy(g_hbm.at[srcv.at[i]], rows, sem).wait()
        pltpu.sync_copy(rows, acc.at[dstv.at[i]], add=True)

    plsc.subcore_barrier()
    for t in range(RPT // CK):
        pltpu.sync_copy(acc.at[pl.ds(s * RPT + t * CK, CK)], rows)
        pltpu.sync_copy(rows, accp_hbm.at[c, pl.ds(s * RPT + t * CK, CK)])
    pltpu.sync_copy(acc.at[pl.ds(s * RPT + RPT - RPT % CK, RPT % CK)],
                    rows.at[pl.ds(0, RPT % CK)])
    pltpu.sync_copy(rows.at[pl.ds(0, RPT % CK)],
                    accp_hbm.at[c, pl.ds(s * RPT + RPT - RPT % CK, RPT % CK)])


@functools.cache
def _edge_call():
    return pl.kernel(
        _edge_body,
        out_type=jax.ShapeDtypeStruct((NC, NP, D), jnp.float32),
        mesh=_mesh(),
        scratch_types=[
            pltpu.VMEM((NCH, CK), jnp.int32),
            pltpu.VMEM((NCH, CK), jnp.int32),
            pltpu.VMEM((CK, D), jnp.float32),
            pltpu.VMEM_SHARED((NP, D), jnp.float32),
            pltpu.SemaphoreType.DMA,
        ],
    )


def _k1_body(x_ref, degp_ref, w1, b1, w2, b2, wc, g_ref, dinv_ref):
    h = jnp.maximum(
        jnp.dot(x_ref[...], w1[...], preferred_element_type=jnp.float32)
        + b1[...], 0.0)
    h = jnp.dot(h, w2[...], preferred_element_type=jnp.float32) + b2[...]
    deg = degp_ref[0, :, 0:1] + degp_ref[1, :, 0:1] + 1.0
    dinv = lax.rsqrt(deg)
    dinv_ref[...] = dinv
    g_ref[...] = jnp.dot(h, wc[...], preferred_element_type=jnp.float32) * dinv


def _full(shape):
    return pl.BlockSpec(shape, lambda *_: tuple(0 for _ in shape))


_k1_call = pl.pallas_call(
    _k1_body,
    grid=(NRB,),
    in_specs=[
        pl.BlockSpec((RB, D), lambda i: (i, 0)),
        pl.BlockSpec((NC, RB, D), lambda i: (0, i, 0)),
        _full((D, D)),
        _full((1, D)),
        _full((D, D)),
        _full((1, D)),
        _full((D, D)),
    ],
    out_specs=[
        pl.BlockSpec((RB, D), lambda i: (i, 0)),
        pl.BlockSpec((RB, 1), lambda i: (i, 0)),
    ],
    out_shape=[
        jax.ShapeDtypeStruct((N, D), jnp.float32),
        jax.ShapeDtypeStruct((N, 1), jnp.float32),
    ],
)


def _k2_body(accp_ref, g_ref, dinv_ref, bc, gamma, beta, wn, out_ref, zs,
             stats):
    p = pl.program_id(0)
    i = pl.program_id(1)
    dinv = dinv_ref[...]

    @pl.when(p == 0)
    def _pass0():
        z = dinv * (accp_ref[0] + accp_ref[1] + g_ref[...]) + bc[...]
        zs[pl.ds(i * RB, RB)] = z

        @pl.when(i == 0)
        def _init():
            stats[...] = jnp.zeros_like(stats)

        stats[0:1] = stats[0:1] + jnp.sum(z, axis=0, keepdims=True)
        stats[1:2] = stats[1:2] + jnp.sum(z * z, axis=0, keepdims=True)

    @pl.when(p == 1)
    def _pass1():
        z = zs[pl.ds(i * RB, RB)]
        mean = stats[0:1] / N
        var = stats[1:2] / N - mean * mean
        hn = jnp.maximum(
            (z - mean) * lax.rsqrt(var + EPS) * gamma[...] + beta[...], 0.0)
        out_ref[...] = jnp.dot(
            hn, wn[...], preferred_element_type=jnp.float32) * dinv


_k2_call = pl.pallas_call(
    _k2_body,
    grid=(2, NRB),
    in_specs=[
        pl.BlockSpec((NC, RB, D), lambda p, i: (0, i, 0)),
        pl.BlockSpec((RB, D), lambda p, i: (i, 0)),
        pl.BlockSpec((RB, 1), lambda p, i: (i, 0)),
        _full((1, D)),
        _full((1, D)),
        _full((1, D)),
        _full((D, D)),
    ],
    out_specs=pl.BlockSpec((RB, D), lambda p, i: (i, 0)),
    out_shape=jax.ShapeDtypeStruct((N, D), jnp.float32),
    scratch_shapes=[
        pltpu.VMEM((N, D), jnp.float32),
        pltpu.VMEM((2, D), jnp.float32),
    ],
)


def _k3_body(accp_ref, g_ref, dinv_ref, batch_ref, bc, gamma, beta, p1, pb1,
             p2, pb2, y_ref, zs, stats):
    p = pl.program_id(0)
    i = pl.program_id(1)
    dinv = dinv_ref[...]

    @pl.when(p == 0)
    def _pass0():
        z = dinv * (accp_ref[0] + accp_ref[1] + g_ref[...]) + bc[...]
        zs[pl.ds(i * RB, RB)] = z

        @pl.when(i == 0)
        def _init():
            stats[...] = jnp.zeros_like(stats)
            y_ref[...] = jnp.zeros_like(y_ref)

        stats[0:1] = stats[0:1] + jnp.sum(z, axis=0, keepdims=True)
        stats[1:2] = stats[1:2] + jnp.sum(z * z, axis=0, keepdims=True)

    @pl.when(p == 1)
    def _pass1():
        z = zs[pl.ds(i * RB, RB)]
        mean = stats[0:1] / N
        var = stats[1:2] / N - mean * mean
        hn = jnp.maximum(
            (z - mean) * lax.rsqrt(var + EPS) * gamma[...] + beta[...], 0.0)
        t = jnp.maximum(
            jnp.dot(hn, p1[...], preferred_element_type=jnp.float32)
            + pb1[...], 0.0)
        o = jnp.dot(t, p2[...], preferred_element_type=jnp.float32) + pb2[...]
        gids = lax.broadcasted_iota(jnp.int32, (1, NG), 1)
        mask = (batch_ref[...] == gids).astype(jnp.float32)
        y_ref[...] += lax.dot_general(
            mask, o, (((0,), (0,)), ((), ())),
            preferred_element_type=jnp.float32)


_k3_call = pl.pallas_call(
    _k3_body,
    grid=(2, NRB),
    in_specs=[
        pl.BlockSpec((NC, RB, D), lambda p, i: (0, i, 0)),
        pl.BlockSpec((RB, D), lambda p, i: (i, 0)),
        pl.BlockSpec((RB, 1), lambda p, i: (i, 0)),
        pl.BlockSpec((RB, 1), lambda p, i: (i, 0)),
        _full((1, D)),
        _full((1, D)),
        _full((1, D)),
        _full((D, DOUT)),
        _full((1, DOUT)),
        _full((DOUT, DOUT)),
        _full((1, DOUT)),
    ],
    out_specs=_full((NG, DOUT)),
    out_shape=jax.ShapeDtypeStruct((NG, DOUT), jnp.float32),
    scratch_shapes=[
        pltpu.VMEM((N, D), jnp.float32),
        pltpu.VMEM((2, D), jnp.float32),
    ],
)


def kernel(x, edge_index, batch, weights):
    w = weights
    W1, b1, W2, b2 = w[0], w[1], w[2], w[3]
    convs = [(w[4 + 4 * l], w[5 + 4 * l], w[6 + 4 * l], w[7 + 4 * l])
             for l in range(DEPTH)]
    P1, pb1, P2, pb2 = w[16], w[17], w[18], w[19]

    src = edge_index[0].reshape(NW, NCH, CK)
    dst = edge_index[1].reshape(NW, NCH, CK)
    ones = jnp.ones((N, D), jnp.float32)

    r = lambda v: v.reshape(1, -1)

    degp = _edge_call()(ones, src, dst)
    g, dinv = _k1_call(x, degp, W1, r(b1), W2, r(b2), convs[0][0])
    for l in range(DEPTH):
        Wc, bc, gamma, beta = convs[l]
        accp = _edge_call()(g, src, dst)
        if l < DEPTH - 1:
            g = _k2_call(accp, g, dinv, r(bc), r(gamma), r(beta),
                         convs[l + 1][0])
        else:
            y = _k3_call(accp, g, dinv, batch.reshape(N, 1), r(bc), r(gamma),
                         r(beta), P1, r(pb1), P2, r(pb2))
    return y


# speedup vs baseline: 15.7555x; 1.1509x over previous
import functools

import jax
import jax.numpy as jnp
from jax import lax
from jax.experimental import pallas as pl
from jax.experimental.pallas import tpu as pltpu
from jax.experimental.pallas import tpu_sc as plsc

N = 10000
E = 320000
D = 128
DOUT = 64
NG = 64
DEPTH = 3
EPS = 1e-5

NC, NS = 2, 16
NW = NC * NS
EPW = E // NW
CK = 125
NCH = EPW // CK
WB = 120
NP = 10112
RPT = NP // NS

RB = 1000
NRB = N // RB


@functools.cache
def _mesh():
    return plsc.VectorSubcoreMesh(
        core_axis_name="c", subcore_axis_name="s",
        num_cores=NC, num_subcores=NS)


def _edge_body(g_hbm, src_hbm, dst_hbm, accp_hbm, srcv, dstv, rows, acc,
               gsem):
    c = lax.axis_index("c")
    s = lax.axis_index("s")
    w = c * NS + s

    @pl.loop(0, CK)
    def _z(i):
        for j in range(D // 16):
            rows[i, pl.ds(j * 16, 16)] = jnp.zeros((16,), jnp.float32)

    for t in range(RPT // CK):
        pltpu.sync_copy(rows, acc.at[pl.ds(s * RPT + t * CK, CK)])
    pltpu.sync_copy(rows.at[pl.ds(0, RPT % CK)],
                    acc.at[pl.ds(s * RPT + RPT - RPT % CK, RPT % CK)])
    plsc.subcore_barrier()
    pltpu.sync_copy(src_hbm.at[w], srcv)
    pltpu.sync_copy(dst_hbm.at[w], dstv)

    @pl.loop(0, NCH)
    def _go(j):
        pltpu.async_copy(g_hbm.at[srcv.at[j]], rows, gsem).wait()
        pltpu.sync_copy(rows, acc.at[dstv.at[j]], add=True)

    plsc.subcore_barrier()
    for t in range(RPT // WB):
        pltpu.sync_copy(acc.at[pl.ds(s * RPT + t * WB, WB)],
                        rows.at[pl.ds(0, WB)])
        pltpu.sync_copy(rows.at[pl.ds(0, WB)],
                        accp_hbm.at[c, pl.ds(s * RPT + t * WB, WB)])
    pltpu.sync_copy(acc.at[pl.ds(s * RPT + RPT - RPT % WB, RPT % WB)],
                    rows.at[pl.ds(0, RPT % WB)])
    pltpu.sync_copy(rows.at[pl.ds(0, RPT % WB)],
                    accp_hbm.at[c, pl.ds(s * RPT + RPT - RPT % WB, RPT % WB)])


@functools.cache
def _edge_call():
    return pl.kernel(
        _edge_body,
        out_type=jax.ShapeDtypeStruct((NC, NP, D), jnp.float32),
        mesh=_mesh(),
        scratch_types=[
            pltpu.VMEM((NCH, CK), jnp.int32),
            pltpu.VMEM((NCH, CK), jnp.int32),
            pltpu.VMEM((CK, D), jnp.float32),
            pltpu.VMEM_SHARED((NP, D), jnp.float32),
            pltpu.SemaphoreType.DMA,
        ],
    )


def _k1_body(x_ref, degp_ref, w1, b1, w2, b2, wc, g_ref, dinv_ref):
    h = jnp.maximum(
        jnp.dot(x_ref[...], w1[...], preferred_element_type=jnp.float32)
        + b1[...], 0.0)
    h = jnp.dot(h, w2[...], preferred_element_type=jnp.float32) + b2[...]
    deg = degp_ref[0, :, 0:1] + degp_ref[1, :, 0:1] + 1.0
    dinv = lax.rsqrt(deg)
    dinv_ref[...] = dinv
    g_ref[...] = jnp.dot(h, wc[...], preferred_element_type=jnp.float32) * dinv


def _full(shape):
    return pl.BlockSpec(shape, lambda *_: tuple(0 for _ in shape))


_k1_call = pl.pallas_call(
    _k1_body,
    grid=(NRB,),
    in_specs=[
        pl.BlockSpec((RB, D), lambda i: (i, 0)),
        pl.BlockSpec((NC, RB, D), lambda i: (0, i, 0)),
        _full((D, D)),
        _full((1, D)),
        _full((D, D)),
        _full((1, D)),
        _full((D, D)),
    ],
    out_specs=[
        pl.BlockSpec((RB, D), lambda i: (i, 0)),
        pl.BlockSpec((RB, 1), lambda i: (i, 0)),
    ],
    out_shape=[
        jax.ShapeDtypeStruct((N, D), jnp.float32),
        jax.ShapeDtypeStruct((N, 1), jnp.float32),
    ],
)


def _k2_body(accp_ref, g_ref, dinv_ref, bc, gamma, beta, wn, out_ref, zs,
             stats):
    p = pl.program_id(0)
    i = pl.program_id(1)
    dinv = dinv_ref[...]

    @pl.when(p == 0)
    def _pass0():
        z = dinv * (accp_ref[0] + accp_ref[1] + g_ref[...]) + bc[...]
        zs[pl.ds(i * RB, RB)] = z

        @pl.when(i == 0)
        def _init():
            stats[...] = jnp.zeros_like(stats)

        stats[0:1] = stats[0:1] + jnp.sum(z, axis=0, keepdims=True)
        stats[1:2] = stats[1:2] + jnp.sum(z * z, axis=0, keepdims=True)

    @pl.when(p == 1)
    def _pass1():
        z = zs[pl.ds(i * RB, RB)]
        mean = stats[0:1] / N
        var = stats[1:2] / N - mean * mean
        hn = jnp.maximum(
            (z - mean) * lax.rsqrt(var + EPS) * gamma[...] + beta[...], 0.0)
        out_ref[...] = jnp.dot(
            hn, wn[...], preferred_element_type=jnp.float32) * dinv


_k2_call = pl.pallas_call(
    _k2_body,
    grid=(2, NRB),
    in_specs=[
        pl.BlockSpec((NC, RB, D), lambda p, i: (0, i, 0)),
        pl.BlockSpec((RB, D), lambda p, i: (i, 0)),
        pl.BlockSpec((RB, 1), lambda p, i: (i, 0)),
        _full((1, D)),
        _full((1, D)),
        _full((1, D)),
        _full((D, D)),
    ],
    out_specs=pl.BlockSpec((RB, D), lambda p, i: (i, 0)),
    out_shape=jax.ShapeDtypeStruct((N, D), jnp.float32),
    scratch_shapes=[
        pltpu.VMEM((N, D), jnp.float32),
        pltpu.VMEM((2, D), jnp.float32),
    ],
)


def _k3_body(accp_ref, g_ref, dinv_ref, batch_ref, bc, gamma, beta, p1, pb1,
             p2, pb2, y_ref, zs, stats):
    p = pl.program_id(0)
    i = pl.program_id(1)
    dinv = dinv_ref[...]

    @pl.when(p == 0)
    def _pass0():
        z = dinv * (accp_ref[0] + accp_ref[1] + g_ref[...]) + bc[...]
        zs[pl.ds(i * RB, RB)] = z

        @pl.when(i == 0)
        def _init():
            stats[...] = jnp.zeros_like(stats)
            y_ref[...] = jnp.zeros_like(y_ref)

        stats[0:1] = stats[0:1] + jnp.sum(z, axis=0, keepdims=True)
        stats[1:2] = stats[1:2] + jnp.sum(z * z, axis=0, keepdims=True)

    @pl.when(p == 1)
    def _pass1():
        z = zs[pl.ds(i * RB, RB)]
        mean = stats[0:1] / N
        var = stats[1:2] / N - mean * mean
        hn = jnp.maximum(
            (z - mean) * lax.rsqrt(var + EPS) * gamma[...] + beta[...], 0.0)
        t = jnp.maximum(
            jnp.dot(hn, p1[...], preferred_element_type=jnp.float32)
            + pb1[...], 0.0)
        o = jnp.dot(t, p2[...], preferred_element_type=jnp.float32) + pb2[...]
        gids = lax.broadcasted_iota(jnp.int32, (1, NG), 1)
        mask = (batch_ref[...] == gids).astype(jnp.float32)
        y_ref[...] += lax.dot_general(
            mask, o, (((0,), (0,)), ((), ())),
            preferred_element_type=jnp.float32)


_k3_call = pl.pallas_call(
    _k3_body,
    grid=(2, NRB),
    in_specs=[
        pl.BlockSpec((NC, RB, D), lambda p, i: (0, i, 0)),
        pl.BlockSpec((RB, D), lambda p, i: (i, 0)),
        pl.BlockSpec((RB, 1), lambda p, i: (i, 0)),
        pl.BlockSpec((RB, 1), lambda p, i: (i, 0)),
        _full((1, D)),
        _full((1, D)),
        _full((1, D)),
        _full((D, DOUT)),
        _full((1, DOUT)),
        _full((DOUT, DOUT)),
        _full((1, DOUT)),
    ],
    out_specs=_full((NG, DOUT)),
    out_shape=jax.ShapeDtypeStruct((NG, DOUT), jnp.float32),
    scratch_shapes=[
        pltpu.VMEM((N, D), jnp.float32),
        pltpu.VMEM((2, D), jnp.float32),
    ],
)


def kernel(x, edge_index, batch, weights):
    w = weights
    W1, b1, W2, b2 = w[0], w[1], w[2], w[3]
    convs = [(w[4 + 4 * l], w[5 + 4 * l], w[6 + 4 * l], w[7 + 4 * l])
             for l in range(DEPTH)]
    P1, pb1, P2, pb2 = w[16], w[17], w[18], w[19]

    src = edge_index[0].reshape(NW, NCH, CK)
    dst = edge_index[1].reshape(NW, NCH, CK)
    ones = jnp.ones((N, D), jnp.float32)

    r = lambda v: v.reshape(1, -1)

    degp = _edge_call()(ones, src, dst)
    g, dinv = _k1_call(x, degp, W1, r(b1), W2, r(b2), convs[0][0])
    for l in range(DEPTH):
        Wc, bc, gamma, beta = convs[l]
        accp = _edge_call()(g, src, dst)
        if l < DEPTH - 1:
            g = _k2_call(accp, g, dinv, r(bc), r(gamma), r(beta),
                         convs[l + 1][0])
        else:
            y = _k3_call(accp, g, dinv, batch.reshape(N, 1), r(bc), r(gamma),
                         r(beta), P1, r(pb1), P2, r(pb2))
    return y
